# initial kernel scaffold (unmeasured)
import functools

import jax
import jax.numpy as jnp
from jax import lax
from jax.experimental import pallas as pl
from jax.experimental.pallas import tpu as pltpu

N_DEV = 4
B, SQ, SKV, HQ, DH = 2, 512, 512, 32, 64
H_LOC = HQ // N_DEV
D_MODEL = 768


def kernel(x, Wq, K_ext, V_ext, Wo):
    def body(x_ref, wq_ref, k_ref, v_ref, wo_ref, out_ref,
             comm_ref, send_sems, recv_sems, ctx_ref):
        my = lax.axis_index("i")
        left = lax.rem(my + N_DEV - 1, N_DEV)
        right = lax.rem(my + 1, N_DEV)

        barrier_sem = pltpu.get_barrier_semaphore()
        for nbr in (left, right):
            pl.semaphore_signal(barrier_sem, inc=1, device_id=(nbr,),
                                device_id_type=pl.DeviceIdType.MESH)
        pl.semaphore_wait(barrier_sem, 2)

        qi = lax.broadcasted_iota(jnp.int32, (SQ, SKV), 0)
        ki = lax.broadcasted_iota(jnp.int32, (SQ, SKV), 1)
        mask = (jnp.abs(qi - ki) <= 128) | (ki < 32) | (qi < 32)

        head0 = my * H_LOC
        for b in range(B):
            q = jnp.dot(x_ref[b], wq_ref[...],
                        preferred_element_type=jnp.float32)
            kb = lax.dynamic_slice_in_dim(k_ref[b], head0, H_LOC, axis=1)
            vb = lax.dynamic_slice_in_dim(v_ref[b], head0, H_LOC, axis=1)
            for h in range(H_LOC):
                qh = q[:, h * DH:(h + 1) * DH]
                kh = kb[:, h, :]
                vh = vb[:, h, :]
                s = lax.dot_general(qh, kh, (((1,), (1,)), ((), ())),
                                    preferred_element_type=jnp.float32)
                s = jnp.where(mask, s * 0.125, -1e9)
                m = jnp.max(s, axis=-1, keepdims=True)
                w = jnp.exp(s - m)
                w = w / jnp.sum(w, axis=-1, keepdims=True)
                ctx_ref[b, :, h * DH:(h + 1) * DH] = jnp.dot(
                    w, vh, preferred_element_type=jnp.float32)
            partial = jnp.dot(ctx_ref[b], wo_ref[...],
                              preferred_element_type=jnp.float32)
            out_ref[b] = partial
            comm_ref[0, b] = partial

        for hop in range(N_DEV - 1):
            rdma = pltpu.make_async_remote_copy(
                src_ref=comm_ref.at[hop],
                dst_ref=comm_ref.at[hop + 1],
                send_sem=send_sems.at[hop],
                recv_sem=recv_sems.at[hop],
                device_id=(right,),
                device_id_type=pl.DeviceIdType.MESH,
            )
            rdma.start()
            rdma.wait()
            out_ref[...] += comm_ref[hop + 1]

        @functools.partial(pl.run_scoped, sem=pltpu.SemaphoreType.REGULAR)
        def _(sem):
            for nbr in (left, right):
                pl.semaphore_signal(sem, inc=1, device_id=(nbr,),
                                    device_id_type=pl.DeviceIdType.MESH)
            pl.semaphore_wait(sem, 2)

    out_shape = jax.ShapeDtypeStruct((B, SQ, D_MODEL), jnp.float32)
    return pl.pallas_call(
        body,
        out_shape=out_shape,
        in_specs=[pl.BlockSpec(memory_space=pltpu.VMEM)] * 5,
        out_specs=pl.BlockSpec(memory_space=pltpu.VMEM),
        scratch_shapes=[
            pltpu.VMEM((N_DEV, B, SQ, D_MODEL), jnp.float32),
            pltpu.SemaphoreType.DMA((N_DEV - 1,)),
            pltpu.SemaphoreType.DMA((N_DEV - 1,)),
            pltpu.VMEM((B, SQ, H_LOC * DH), jnp.float32),
        ],
        compiler_params=pltpu.CompilerParams(collective_id=0),
    )(x, Wq, K_ext, V_ext, Wo)


# baseline (device time: 177363 ns/iter reference)
import functools

import jax
import jax.numpy as jnp
from jax import lax
from jax.experimental import pallas as pl
from jax.experimental.pallas import tpu as pltpu

N_DEV = 4
B, SQ, SKV, HQ, DH = 2, 512, 512, 32, 64
H_LOC = HQ // N_DEV
D_MODEL = 768


def kernel(x, Wq, K_ext, V_ext, Wo):
    def body(x_ref, wq_ref, k_ref, v_ref, wo_ref, out_ref,
             comm_ref, send_sems, recv_sems, ctx_ref):
        my = lax.axis_index("i")
        left = lax.rem(my + N_DEV - 1, N_DEV)
        right = lax.rem(my + 1, N_DEV)

        barrier_sem = pltpu.get_barrier_semaphore()
        for nbr in (left, right):
            pl.semaphore_signal(barrier_sem, inc=1, device_id=(nbr,),
                                device_id_type=pl.DeviceIdType.MESH)
        pl.semaphore_wait(barrier_sem, 2)

        qi = lax.broadcasted_iota(jnp.int32, (SQ, SKV), 0)
        ki = lax.broadcasted_iota(jnp.int32, (SQ, SKV), 1)
        mask = (jnp.abs(qi - ki) <= 128) | (ki < 32) | (qi < 32)

        head0 = my * H_LOC
        for b in range(B):
            q = jnp.dot(x_ref[b], wq_ref[...],
                        preferred_element_type=jnp.float32)
            kb = k_ref[b, :, pl.ds(head0, H_LOC), :]
            vb = v_ref[b, :, pl.ds(head0, H_LOC), :]
            for h in range(H_LOC):
                qh = q[:, h * DH:(h + 1) * DH]
                kh = kb[:, h, :]
                vh = vb[:, h, :]
                s = lax.dot_general(qh, kh, (((1,), (1,)), ((), ())),
                                    preferred_element_type=jnp.float32)
                s = jnp.where(mask, s * 0.125, -1e9)
                m = jnp.max(s, axis=-1, keepdims=True)
                w = jnp.exp(s - m)
                w = w / jnp.sum(w, axis=-1, keepdims=True)
                ctx_ref[b, :, h * DH:(h + 1) * DH] = jnp.dot(
                    w, vh, preferred_element_type=jnp.float32)
            partial = jnp.dot(ctx_ref[b], wo_ref[...],
                              preferred_element_type=jnp.float32)
            out_ref[b] = partial
            comm_ref[0, b] = partial

        for hop in range(N_DEV - 1):
            rdma = pltpu.make_async_remote_copy(
                src_ref=comm_ref.at[hop],
                dst_ref=comm_ref.at[hop + 1],
                send_sem=send_sems.at[hop],
                recv_sem=recv_sems.at[hop],
                device_id=(right,),
                device_id_type=pl.DeviceIdType.MESH,
            )
            rdma.start()
            rdma.wait()
            out_ref[...] += comm_ref[hop + 1]

        @functools.partial(pl.run_scoped, sem=pltpu.SemaphoreType.REGULAR)
        def _(sem):
            for nbr in (left, right):
                pl.semaphore_signal(sem, inc=1, device_id=(nbr,),
                                    device_id_type=pl.DeviceIdType.MESH)
            pl.semaphore_wait(sem, 2)

    out_shape = jax.ShapeDtypeStruct((B, SQ, D_MODEL), jnp.float32)
    return pl.pallas_call(
        body,
        out_shape=out_shape,
        in_specs=[pl.BlockSpec(memory_space=pltpu.VMEM)] * 5,
        out_specs=pl.BlockSpec(memory_space=pltpu.VMEM),
        scratch_shapes=[
            pltpu.VMEM((N_DEV, B, SQ, D_MODEL), jnp.float32),
            pltpu.SemaphoreType.DMA((N_DEV - 1,)),
            pltpu.SemaphoreType.DMA((N_DEV - 1,)),
            pltpu.VMEM((B, SQ, H_LOC * DH), jnp.float32),
        ],
        compiler_params=pltpu.CompilerParams(
            collective_id=0, vmem_limit_bytes=100 * 1024 * 1024),
    )(x, Wq, K_ext, V_ext, Wo)


# device time: 82188 ns/iter; 2.1580x vs baseline; 2.1580x over previous
import functools

import jax
import jax.numpy as jnp
from jax import lax
from jax.experimental import pallas as pl
from jax.experimental.pallas import tpu as pltpu

N_DEV = 4
B, SQ, SKV, HQ, DH = 2, 512, 512, 32, 64
H_LOC = HQ // N_DEV
D_MODEL = 768
HALF = SQ // 2


def kernel(x, Wq, K_ext, V_ext, Wo):
    def body(x_ref, wq_ref, k_ref, v_ref, wo_ref, out_ref,
             rt_comm, rt_send, rt_recv, lt_comm, lt_send, lt_recv,
             kscr, vscr, kv_sem):
        my = lax.axis_index("i")
        left = lax.rem(my + N_DEV - 1, N_DEV)
        right = lax.rem(my + 1, N_DEV)
        head0 = my * H_LOC

        kv_copies = []
        for b in range(B):
            for src, dst in ((k_ref, kscr), (v_ref, vscr)):
                cp = pltpu.make_async_copy(
                    src.at[b, pl.ds(head0, H_LOC)], dst.at[b], kv_sem)
                cp.start()
                kv_copies.append(cp)

        barrier_sem = pltpu.get_barrier_semaphore()
        for nbr in (left, right):
            pl.semaphore_signal(barrier_sem, inc=1, device_id=(nbr,),
                                device_id_type=pl.DeviceIdType.MESH)
        pl.semaphore_wait(barrier_sem, 2)

        qi = lax.broadcasted_iota(jnp.int32, (SQ, SKV), 0)
        ki = lax.broadcasted_iota(jnp.int32, (SQ, SKV), 1)
        mask = (jnp.abs(qi - ki) <= 128) | (ki < 32) | (qi < 32)
        bias = jnp.where(mask, 0.0, -1e9).astype(jnp.float32)

        wq_bf = wq_ref[...].astype(jnp.bfloat16)
        wo_bf = wo_ref[...].astype(jnp.bfloat16)
        for cp in kv_copies:
            cp.wait()
        for b in range(B):
            q = jnp.dot(x_ref[b].astype(jnp.bfloat16), wq_bf,
                        preferred_element_type=jnp.float32)
            q = q.astype(jnp.bfloat16)
            partial = jnp.zeros((SQ, D_MODEL), jnp.float32)
            for h in range(H_LOC):
                qh = q[:, h * DH:(h + 1) * DH]
                kht = kscr[b, h].astype(jnp.bfloat16)
                vht = vscr[b, h].astype(jnp.bfloat16)
                s = jnp.dot(qh, kht, preferred_element_type=jnp.float32)
                w = jnp.exp(s * 0.125 + bias).astype(jnp.bfloat16)
                zinv = 1.0 / jnp.sum(w.astype(jnp.float32),
                                     axis=-1, keepdims=True)
                ctxh = lax.dot_general(w, vht, (((1,), (1,)), ((), ())),
                                       preferred_element_type=jnp.float32)
                partial += jnp.dot(
                    (ctxh * zinv).astype(jnp.bfloat16),
                    wo_bf[h * DH:(h + 1) * DH, :],
                    preferred_element_type=jnp.float32)
            out_ref[b] = partial
            rt_comm[0, b] = partial[:HALF, :]
            lt_comm[0, b] = partial[HALF:, :]

        for hop in range(N_DEV - 1):
            r = pltpu.make_async_remote_copy(
                src_ref=rt_comm.at[hop], dst_ref=rt_comm.at[hop + 1],
                send_sem=rt_send.at[hop], recv_sem=rt_recv.at[hop],
                device_id=(right,), device_id_type=pl.DeviceIdType.MESH,
            )
            l = pltpu.make_async_remote_copy(
                src_ref=lt_comm.at[hop], dst_ref=lt_comm.at[hop + 1],
                send_sem=lt_send.at[hop], recv_sem=lt_recv.at[hop],
                device_id=(left,), device_id_type=pl.DeviceIdType.MESH,
            )
            r.start()
            l.start()
            r.wait_recv()
            out_ref[:, :HALF, :] += rt_comm[hop + 1]
            l.wait_recv()
            out_ref[:, HALF:, :] += lt_comm[hop + 1]
            r.wait_send()
            l.wait_send()

        @functools.partial(pl.run_scoped, sem=pltpu.SemaphoreType.REGULAR)
        def _(sem):
            for nbr in (left, right):
                pl.semaphore_signal(sem, inc=1, device_id=(nbr,),
                                    device_id_type=pl.DeviceIdType.MESH)
            pl.semaphore_wait(sem, 2)

    out_shape = jax.ShapeDtypeStruct((B, SQ, D_MODEL), jnp.float32)
    return pl.pallas_call(
        body,
        out_shape=out_shape,
        in_specs=[
            pl.BlockSpec(memory_space=pltpu.VMEM),
            pl.BlockSpec(memory_space=pltpu.VMEM),
            pl.BlockSpec(memory_space=pltpu.MemorySpace.HBM),
            pl.BlockSpec(memory_space=pltpu.MemorySpace.HBM),
            pl.BlockSpec(memory_space=pltpu.VMEM),
        ],
        out_specs=pl.BlockSpec(memory_space=pltpu.VMEM),
        scratch_shapes=[
            pltpu.VMEM((N_DEV, B, HALF, D_MODEL), jnp.float32),
            pltpu.SemaphoreType.DMA((N_DEV - 1,)),
            pltpu.SemaphoreType.DMA((N_DEV - 1,)),
            pltpu.VMEM((N_DEV, B, HALF, D_MODEL), jnp.float32),
            pltpu.SemaphoreType.DMA((N_DEV - 1,)),
            pltpu.SemaphoreType.DMA((N_DEV - 1,)),
            pltpu.VMEM((B, H_LOC, DH, SKV), jnp.float32),
            pltpu.VMEM((B, H_LOC, DH, SKV), jnp.float32),
            pltpu.SemaphoreType.DMA,
        ],
        compiler_params=pltpu.CompilerParams(
            collective_id=0, vmem_limit_bytes=100 * 1024 * 1024),
    )(x, Wq, jnp.transpose(K_ext, (0, 2, 3, 1)),
      jnp.transpose(V_ext, (0, 2, 3, 1)), Wo)


# device time: 60992 ns/iter; 2.9080x vs baseline; 1.3475x over previous
import functools

import jax
import jax.numpy as jnp
from jax import lax
from jax.experimental import pallas as pl
from jax.experimental.pallas import tpu as pltpu

N_DEV = 4
B, SQ, SKV, HQ, DH = 2, 512, 512, 32, 64
H_LOC = HQ // N_DEV
D_MODEL = 768
HALF = SQ // 2
CHUNK = HALF // N_DEV


def kernel(x, Wq, K_ext, V_ext, Wo):
    def body(x_ref, wq_ref, k_ref, v_ref, wo_ref, out_ref,
             rt_acc, rt_stage, lt_acc, lt_stage,
             r_rs_send, r_rs_recv, r_ag_send, r_ag_recv,
             l_rs_send, l_rs_recv, l_ag_send, l_ag_recv,
             kscr, vscr, kv_sem, xv, x_sem):
        my = lax.axis_index("i")
        left = lax.rem(my + N_DEV - 1, N_DEV)
        right = lax.rem(my + 1, N_DEV)
        head0 = my * H_LOC

        x_cp = pltpu.make_async_copy(x_ref, xv, x_sem)
        x_cp.start()
        kv_copies = []
        for b in range(B):
            for src, dst in ((k_ref, kscr), (v_ref, vscr)):
                cp = pltpu.make_async_copy(
                    src.at[b, pl.ds(head0, H_LOC)], dst.at[b], kv_sem)
                cp.start()
                kv_copies.append(cp)

        barrier_sem = pltpu.get_barrier_semaphore()
        for nbr in (left, right):
            pl.semaphore_signal(barrier_sem, inc=1, device_id=(nbr,),
                                device_id_type=pl.DeviceIdType.MESH)
        pl.semaphore_wait(barrier_sem, 2)

        qi = lax.broadcasted_iota(jnp.int32, (SQ, SKV), 0)
        ki = lax.broadcasted_iota(jnp.int32, (SQ, SKV), 1)
        mask = (jnp.abs(qi - ki) <= 128) | (ki < 32) | (qi < 32)
        bias = jnp.where(mask, 0.0, -1e9).astype(jnp.float32)

        wq_bf = wq_ref[...].astype(jnp.bfloat16)
        wo_bf = wo_ref[...].astype(jnp.bfloat16)
        for cp in kv_copies:
            cp.wait()
        x_cp.wait()
        for b in range(B):
            q = jnp.dot(xv[b].astype(jnp.bfloat16), wq_bf,
                        preferred_element_type=jnp.float32)
            q = q.astype(jnp.bfloat16)
            partial = jnp.zeros((SQ, D_MODEL), jnp.float32)
            for h in range(H_LOC):
                qh = q[:, h * DH:(h + 1) * DH]
                kht = kscr[b, h].astype(jnp.bfloat16)
                vht = vscr[b, h].astype(jnp.bfloat16)
                s = jnp.dot(qh, kht, preferred_element_type=jnp.float32)
                w = jnp.exp(s * 0.125 + bias).astype(jnp.bfloat16)
                zinv = 1.0 / jnp.sum(w.astype(jnp.float32),
                                     axis=-1, keepdims=True)
                ctxh = lax.dot_general(w, vht, (((1,), (1,)), ((), ())),
                                       preferred_element_type=jnp.float32)
                partial += jnp.dot(
                    (ctxh * zinv).astype(jnp.bfloat16),
                    wo_bf[h * DH:(h + 1) * DH, :],
                    preferred_element_type=jnp.float32)
            rt_acc[b] = partial[:HALF, :]
            lt_acc[b] = partial[HALF:, :]

        def rows(c):
            return pl.ds(pl.multiple_of(c * CHUNK, CHUNK), CHUNK)

        pending = []

        for s in range(N_DEV - 1):
            cs_r = lax.rem(my - s + 2 * N_DEV, N_DEV)
            cr_r = lax.rem(my - s - 1 + 2 * N_DEV, N_DEV)
            cs_l = lax.rem(my + s, N_DEV)
            cr_l = lax.rem(my + s + 1, N_DEV)
            r = pltpu.make_async_remote_copy(
                src_ref=rt_acc.at[:, rows(cs_r), :],
                dst_ref=rt_stage.at[s],
                send_sem=r_rs_send.at[s], recv_sem=r_rs_recv.at[s],
                device_id=(right,), device_id_type=pl.DeviceIdType.MESH,
            )
            l = pltpu.make_async_remote_copy(
                src_ref=lt_acc.at[:, rows(cs_l), :],
                dst_ref=lt_stage.at[s],
                send_sem=l_rs_send.at[s], recv_sem=l_rs_recv.at[s],
                device_id=(left,), device_id_type=pl.DeviceIdType.MESH,
            )
            r.start()
            l.start()
            r.wait_recv()
            rt_acc[:, rows(cr_r), :] += rt_stage[s]
            l.wait_recv()
            lt_acc[:, rows(cr_l), :] += lt_stage[s]
            pending += [r, l]

        cfin_r = lax.rem(my + 1, N_DEV)
        cfin_l = lax.rem(my - 1 + N_DEV, N_DEV)
        out_ref[:, rows(cfin_r), :] = rt_acc[:, rows(cfin_r), :]
        lrow_fin = pl.ds(pl.multiple_of(HALF + cfin_l * CHUNK, CHUNK), CHUNK)
        out_ref[:, lrow_fin, :] = lt_acc[:, rows(cfin_l), :]

        for s in range(N_DEV - 1):
            ca_r = lax.rem(my + 1 - s + 2 * N_DEV, N_DEV)
            ca_l = lax.rem(my - 1 + s + 2 * N_DEV, N_DEV)
            cr_r = lax.rem(my - s + 2 * N_DEV, N_DEV)
            cr_l = lax.rem(my + s, N_DEV)
            lrow_a = pl.ds(pl.multiple_of(HALF + ca_l * CHUNK, CHUNK), CHUNK)
            r = pltpu.make_async_remote_copy(
                src_ref=out_ref.at[:, rows(ca_r), :],
                dst_ref=out_ref.at[:, rows(ca_r), :],
                send_sem=r_ag_send.at[s], recv_sem=r_ag_recv.at[s],
                device_id=(right,), device_id_type=pl.DeviceIdType.MESH,
            )
            l = pltpu.make_async_remote_copy(
                src_ref=out_ref.at[:, lrow_a, :],
                dst_ref=out_ref.at[:, lrow_a, :],
                send_sem=l_ag_send.at[s], recv_sem=l_ag_recv.at[s],
                device_id=(left,), device_id_type=pl.DeviceIdType.MESH,
            )
            r.start()
            l.start()
            r.wait_recv()
            l.wait_recv()
            pending += [r, l]

        for p in pending:
            p.wait_send()

        @functools.partial(pl.run_scoped, sem=pltpu.SemaphoreType.REGULAR)
        def _(sem):
            for nbr in (left, right):
                pl.semaphore_signal(sem, inc=1, device_id=(nbr,),
                                    device_id_type=pl.DeviceIdType.MESH)
            pl.semaphore_wait(sem, 2)

    out_shape = jax.ShapeDtypeStruct((B, SQ, D_MODEL), jnp.float32)
    nsteps = N_DEV - 1
    return pl.pallas_call(
        body,
        out_shape=out_shape,
        in_specs=[
            pl.BlockSpec(memory_space=pltpu.MemorySpace.HBM),
            pl.BlockSpec(memory_space=pltpu.VMEM),
            pl.BlockSpec(memory_space=pltpu.MemorySpace.HBM),
            pl.BlockSpec(memory_space=pltpu.MemorySpace.HBM),
            pl.BlockSpec(memory_space=pltpu.VMEM),
        ],
        out_specs=pl.BlockSpec(memory_space=pltpu.VMEM),
        scratch_shapes=[
            pltpu.VMEM((B, HALF, D_MODEL), jnp.float32),
            pltpu.VMEM((nsteps, B, CHUNK, D_MODEL), jnp.float32),
            pltpu.VMEM((B, HALF, D_MODEL), jnp.float32),
            pltpu.VMEM((nsteps, B, CHUNK, D_MODEL), jnp.float32),
            pltpu.SemaphoreType.DMA((nsteps,)),
            pltpu.SemaphoreType.DMA((nsteps,)),
            pltpu.SemaphoreType.DMA((nsteps,)),
            pltpu.SemaphoreType.DMA((nsteps,)),
            pltpu.SemaphoreType.DMA((nsteps,)),
            pltpu.SemaphoreType.DMA((nsteps,)),
            pltpu.SemaphoreType.DMA((nsteps,)),
            pltpu.SemaphoreType.DMA((nsteps,)),
            pltpu.VMEM((B, H_LOC, DH, SKV), jnp.float32),
            pltpu.VMEM((B, H_LOC, DH, SKV), jnp.float32),
            pltpu.SemaphoreType.DMA,
            pltpu.VMEM((B, SQ, D_MODEL), jnp.float32),
            pltpu.SemaphoreType.DMA,
        ],
        compiler_params=pltpu.CompilerParams(
            collective_id=0, vmem_limit_bytes=100 * 1024 * 1024),
    )(x, Wq, jnp.transpose(K_ext, (0, 2, 3, 1)),
      jnp.transpose(V_ext, (0, 2, 3, 1)), Wo)
